# Initial kernel scaffold; baseline (speedup 1.0000x reference)
import jax
import jax.numpy as jnp
from jax.experimental import pallas as pl


def kernel(x):
    raise NotImplementedError("write your pallas kernel here")


# speedup vs baseline: 1.7392x; 1.7392x over previous
import jax
import jax.numpy as jnp
from jax.experimental import pallas as pl


def _gather_kernel(x_ref, o_ref):
    o_ref[0, :] = x_ref[2, :]
    o_ref[1, :] = x_ref[4, :]
    o_ref[2, :] = x_ref[5, :]


def kernel(x):
    return pl.pallas_call(
        _gather_kernel,
        out_shape=jax.ShapeDtypeStruct((3, 128), x.dtype),
        grid=(1,),
        in_specs=[pl.BlockSpec((8, 128), lambda i: (0, 0))],
        out_specs=pl.BlockSpec((3, 128), lambda i: (0, 0)),
    )(x)
